# baseline (speedup 1.0000x reference)
import jax
import jax.numpy as jnp
from jax.experimental import pallas as pl
from jax.experimental.pallas import tpu as pltpu

_B = 65536
_N = 24
_THRESH_SQ = 0.25
_NEG4 = -4.0

_LANES = 1152
_ROWS = _B * 72 // _LANES
_SBR = 128
_GRID = _ROWS // _SBR


def _collision_body(x_ref, e_ref, c_ref, xt_ref):
    xt_ref[...] = jnp.transpose(x_ref[...]).reshape(16, 72, _SBR)
    acc_e = jnp.zeros((16, _SBR), jnp.float32)
    acc_c = jnp.zeros((16, _SBR), jnp.float32)
    for i in range(1, _N):
        xi = xt_ref[:, 3 * i, :]
        yi = xt_ref[:, 3 * i + 1, :]
        zi = xt_ref[:, 3 * i + 2, :]
        for j in range(i + 1, _N):
            if i == 2 and j == 3:
                continue
            dx = xi - xt_ref[:, 3 * j, :]
            dy = yi - xt_ref[:, 3 * j + 1, :]
            dz = zi - xt_ref[:, 3 * j + 2, :]
            sq = dx * dx + dy * dy + dz * dz
            sel = sq < _THRESH_SQ
            e = jnp.exp(sq * _NEG4)
            acc_e = acc_e + jnp.where(sel, e, 0.0)
            acc_c = acc_c + jnp.where(sel, 1.0, 0.0)
    e_ref[...] = jnp.sum(acc_e, axis=0).reshape(1, 1, _SBR)
    c_ref[...] = jnp.sum(acc_c, axis=0).reshape(1, 1, _SBR)


def kernel(pos):
    x = pos.reshape(_ROWS, _LANES)

    e_part, c_part = pl.pallas_call(
        _collision_body,
        grid=(_GRID,),
        in_specs=[
            pl.BlockSpec((_SBR, _LANES), lambda g: (g, 0)),
        ],
        out_specs=[
            pl.BlockSpec((1, 1, _SBR), lambda g: (g, 0, 0)),
            pl.BlockSpec((1, 1, _SBR), lambda g: (g, 0, 0)),
        ],
        out_shape=[
            jax.ShapeDtypeStruct((_GRID, 1, _SBR), jnp.float32),
            jax.ShapeDtypeStruct((_GRID, 1, _SBR), jnp.float32),
        ],
        scratch_shapes=[pltpu.VMEM((16, 72, _SBR), jnp.float32)],
        compiler_params=pltpu.CompilerParams(
            dimension_semantics=("parallel",),
        ),
    )(x)

    se = jnp.sum(e_part)
    cnt = jnp.sum(c_part)
    total = jnp.where(cnt > 0, se / jnp.maximum(cnt, 1.0), 0.0)
    return total + 1e-6


# speedup vs baseline: 30.2724x; 30.2724x over previous
import jax
import jax.numpy as jnp
from jax.experimental import pallas as pl
from jax.experimental.pallas import tpu as pltpu

_B = 65536
_N = 24
_THRESH_SQ = 0.25
_NEG4 = -4.0

_LANES = 128
_BROW = _B // _LANES
_SB = 32
_GRID = _BROW // _SB


def _collision_body(x_ref, e_ref, c_ref, t2_ref, d_ref):
    t2_ref[...] = jnp.transpose(x_ref[...], (0, 2, 1)).reshape(_SB * 72, _LANES)
    for r in range(72):
        d_ref[r] = t2_ref[pl.Slice(r, _SB, 72), :]
    acc_e = jnp.zeros((_SB, _LANES), jnp.float32)
    acc_c = jnp.zeros((_SB, _LANES), jnp.float32)
    for i in range(1, _N):
        xi = d_ref[3 * i]
        yi = d_ref[3 * i + 1]
        zi = d_ref[3 * i + 2]
        for j in range(i + 1, _N):
            if i == 2 and j == 3:
                continue
            dx = xi - d_ref[3 * j]
            dy = yi - d_ref[3 * j + 1]
            dz = zi - d_ref[3 * j + 2]
            sq = dx * dx + dy * dy + dz * dz
            sel = sq < _THRESH_SQ
            e = jnp.exp(sq * _NEG4)
            acc_e = acc_e + jnp.where(sel, e, 0.0)
            acc_c = acc_c + jnp.where(sel, 1.0, 0.0)
    e_ref[...] = jnp.sum(acc_e, axis=0).reshape(1, 1, _LANES)
    c_ref[...] = jnp.sum(acc_c, axis=0).reshape(1, 1, _LANES)


def kernel(pos):
    x = pos.reshape(_BROW, _LANES, 72)

    e_part, c_part = pl.pallas_call(
        _collision_body,
        grid=(_GRID,),
        in_specs=[
            pl.BlockSpec((_SB, _LANES, 72), lambda g: (g, 0, 0)),
        ],
        out_specs=[
            pl.BlockSpec((1, 1, _LANES), lambda g: (g, 0, 0)),
            pl.BlockSpec((1, 1, _LANES), lambda g: (g, 0, 0)),
        ],
        out_shape=[
            jax.ShapeDtypeStruct((_GRID, 1, _LANES), jnp.float32),
            jax.ShapeDtypeStruct((_GRID, 1, _LANES), jnp.float32),
        ],
        scratch_shapes=[
            pltpu.VMEM((_SB * 72, _LANES), jnp.float32),
            pltpu.VMEM((72, _SB, _LANES), jnp.float32),
        ],
        compiler_params=pltpu.CompilerParams(
            dimension_semantics=("parallel",),
        ),
    )(x)

    se = jnp.sum(e_part)
    cnt = jnp.sum(c_part)
    total = jnp.where(cnt > 0, se / jnp.maximum(cnt, 1.0), 0.0)
    return total + 1e-6


# speedup vs baseline: 95.1308x; 3.1425x over previous
import jax
import jax.numpy as jnp
from jax.experimental import pallas as pl
from jax.experimental.pallas import tpu as pltpu

_B = 65536
_N = 24
_THRESH_SQ = 0.25
_NEG4 = -4.0

_LANES = 128
_BL = 8192
_SUB = _BL // _LANES
_GRID = _B // _BL


def _collision_body(x_ref, e_ref, c_ref, t2_ref, d_ref):
    t2 = jnp.swapaxes(
        jnp.transpose(x_ref[...]).reshape(_SUB, _LANES, 72), 1, 2
    )
    t2_ref[...] = t2.reshape(_SUB * 72, _LANES)
    for r in range(72):
        d_ref[r] = t2_ref[pl.Slice(r, _SUB, 72), :]
    acc_e = jnp.zeros((_SUB, _LANES), jnp.float32)
    acc_c = jnp.zeros((_SUB, _LANES), jnp.float32)
    for i in range(1, _N):
        xi = d_ref[i]
        yi = d_ref[_N + i]
        zi = d_ref[2 * _N + i]
        for j in range(i + 1, _N):
            if i == 2 and j == 3:
                continue
            dx = xi - d_ref[j]
            dy = yi - d_ref[_N + j]
            dz = zi - d_ref[2 * _N + j]
            sq = dx * dx + dy * dy + dz * dz
            sel = sq < _THRESH_SQ
            e = jnp.exp(sq * _NEG4)
            acc_e = acc_e + jnp.where(sel, e, 0.0)
            acc_c = acc_c + jnp.where(sel, 1.0, 0.0)
    e_ref[...] = jnp.sum(acc_e, axis=0).reshape(1, 1, _LANES)
    c_ref[...] = jnp.sum(acc_c, axis=0).reshape(1, 1, _LANES)


def kernel(pos):
    xt = pos.transpose(2, 1, 0).reshape(3 * _N, _B)

    e_part, c_part = pl.pallas_call(
        _collision_body,
        grid=(_GRID,),
        in_specs=[
            pl.BlockSpec((3 * _N, _BL), lambda g: (0, g)),
        ],
        out_specs=[
            pl.BlockSpec((1, 1, _LANES), lambda g: (g, 0, 0)),
            pl.BlockSpec((1, 1, _LANES), lambda g: (g, 0, 0)),
        ],
        out_shape=[
            jax.ShapeDtypeStruct((_GRID, 1, _LANES), jnp.float32),
            jax.ShapeDtypeStruct((_GRID, 1, _LANES), jnp.float32),
        ],
        scratch_shapes=[
            pltpu.VMEM((_SUB * 72, _LANES), jnp.float32),
            pltpu.VMEM((72, _SUB, _LANES), jnp.float32),
        ],
        compiler_params=pltpu.CompilerParams(
            dimension_semantics=("parallel",),
        ),
    )(xt)

    se = jnp.sum(e_part)
    cnt = jnp.sum(c_part)
    total = jnp.where(cnt > 0, se / jnp.maximum(cnt, 1.0), 0.0)
    return total + 1e-6


# speedup vs baseline: 108.0651x; 1.1360x over previous
import jax
import jax.numpy as jnp
from jax.experimental import pallas as pl
from jax.experimental.pallas import tpu as pltpu

_B = 65536
_N = 24
_THRESH_SQ = 0.25
_NEG4 = -4.0

_LANES = 128
_BL = 8192
_SUB = _BL // _LANES
_GRID = _B // _BL


def _collision_body(x_ref, out_ref, ae_ref, ac_ref, d_ref):
    g = pl.program_id(0)
    for r in range(72):
        d_ref[r] = x_ref[r].reshape(_SUB, _LANES)
    acc_e = jnp.zeros((_SUB, _LANES), jnp.float32)
    acc_c = jnp.zeros((_SUB, _LANES), jnp.float32)
    for i in range(1, _N):
        xi = d_ref[i]
        yi = d_ref[_N + i]
        zi = d_ref[2 * _N + i]
        for j in range(i + 1, _N):
            if i == 2 and j == 3:
                continue
            dx = xi - d_ref[j]
            dy = yi - d_ref[_N + j]
            dz = zi - d_ref[2 * _N + j]
            sq = dx * dx + dy * dy + dz * dz
            sel = sq < _THRESH_SQ
            e = jnp.exp(sq * _NEG4)
            acc_e = acc_e + jnp.where(sel, e, 0.0)
            acc_c = acc_c + jnp.where(sel, 1.0, 0.0)

    @pl.when(g == 0)
    def _init():
        ae_ref[...] = acc_e
        ac_ref[...] = acc_c

    @pl.when(g > 0)
    def _accum():
        ae_ref[...] += acc_e
        ac_ref[...] += acc_c

    @pl.when(g == _GRID - 1)
    def _finalize():
        se = jnp.sum(ae_ref[...])
        cnt = jnp.sum(ac_ref[...])
        total = jnp.where(cnt > 0, se / jnp.maximum(cnt, 1.0), 0.0)
        out_ref[0, 0] = total + 1e-6


def kernel(pos):
    xt = pos.transpose(2, 1, 0).reshape(3 * _N, _B)

    out = pl.pallas_call(
        _collision_body,
        grid=(_GRID,),
        in_specs=[
            pl.BlockSpec((3 * _N, _BL), lambda g: (0, g)),
        ],
        out_specs=pl.BlockSpec(memory_space=pltpu.SMEM),
        out_shape=jax.ShapeDtypeStruct((1, 1), jnp.float32),
        scratch_shapes=[
            pltpu.VMEM((_SUB, _LANES), jnp.float32),
            pltpu.VMEM((_SUB, _LANES), jnp.float32),
            pltpu.VMEM((72, _SUB, _LANES), jnp.float32),
        ],
        compiler_params=pltpu.CompilerParams(
            dimension_semantics=("arbitrary",),
        ),
    )(xt)

    return out.reshape(())


# speedup vs baseline: 151.3156x; 1.4002x over previous
import jax
import jax.numpy as jnp
from jax.experimental import pallas as pl
from jax.experimental.pallas import tpu as pltpu

_B = 65536
_N = 24

_K = -4.0 * 1.4426950408889634
_THK = 0.25 * _K
_S = (-2.0 * _K) ** 0.5

_LANES = 128
_BL = 8192
_SUB = _BL // _LANES
_GRID = _B // _BL


def _collision_body(x_ref, out_ref, ae_ref, ac_ref, d_ref, n_ref):
    g = pl.program_id(0)
    for r in range(72):
        d_ref[r] = x_ref[r].reshape(_SUB, _LANES) * _S
    for i in range(1, _N):
        n_ref[i] = -0.5 * (
            d_ref[i] * d_ref[i]
            + d_ref[_N + i] * d_ref[_N + i]
            + d_ref[2 * _N + i] * d_ref[2 * _N + i]
        )
    acc_e = jnp.zeros((_SUB, _LANES), jnp.float32)
    acc_c = jnp.zeros((_SUB, _LANES), jnp.float32)
    for i in range(1, _N):
        ni = n_ref[i]
        xi = d_ref[i]
        yi = d_ref[_N + i]
        zi = d_ref[2 * _N + i]
        for j in range(i + 1, _N):
            if i == 2 and j == 3:
                continue
            dot = xi * d_ref[j] + yi * d_ref[_N + j] + zi * d_ref[2 * _N + j]
            sqk = (ni + n_ref[j]) + dot
            sel = sqk > _THK
            e = jnp.exp2(sqk)
            acc_e = jnp.where(sel, acc_e + e, acc_e)
            acc_c = jnp.where(sel, acc_c + 1.0, acc_c)

    @pl.when(g == 0)
    def _init():
        ae_ref[...] = acc_e
        ac_ref[...] = acc_c

    @pl.when(g > 0)
    def _accum():
        ae_ref[...] += acc_e
        ac_ref[...] += acc_c

    @pl.when(g == _GRID - 1)
    def _finalize():
        se = jnp.sum(ae_ref[...])
        cnt = jnp.sum(ac_ref[...])
        total = jnp.where(cnt > 0, se / jnp.maximum(cnt, 1.0), 0.0)
        out_ref[0, 0] = total + 1e-6


def kernel(pos):
    xt = pos.transpose(2, 1, 0).reshape(3 * _N, _B)

    out = pl.pallas_call(
        _collision_body,
        grid=(_GRID,),
        in_specs=[
            pl.BlockSpec((3 * _N, _BL), lambda g: (0, g)),
        ],
        out_specs=pl.BlockSpec(memory_space=pltpu.SMEM),
        out_shape=jax.ShapeDtypeStruct((1, 1), jnp.float32),
        scratch_shapes=[
            pltpu.VMEM((_SUB, _LANES), jnp.float32),
            pltpu.VMEM((_SUB, _LANES), jnp.float32),
            pltpu.VMEM((72, _SUB, _LANES), jnp.float32),
            pltpu.VMEM((_N, _SUB, _LANES), jnp.float32),
        ],
        compiler_params=pltpu.CompilerParams(
            dimension_semantics=("arbitrary",),
        ),
    )(xt)

    return out.reshape(())


# speedup vs baseline: 151.5494x; 1.0015x over previous
import jax
import jax.numpy as jnp
from jax.experimental import pallas as pl
from jax.experimental.pallas import tpu as pltpu

_B = 65536
_N = 24

_K = -4.0 * 1.4426950408889634
_THK = 0.25 * _K
_S = (-2.0 * _K) ** 0.5

_LANES = 128
_BL = 8192
_SUB = _BL // _LANES
_GRID = _B // _BL


def _collision_body(x_ref, out_ref, ae_ref, ac_ref, d_ref, n_ref):
    g = pl.program_id(0)
    for r in range(72):
        if r % _N == 0:
            continue
        d_ref[r] = x_ref[r].reshape(_SUB, _LANES) * _S
    for i in range(1, _N):
        n_ref[i] = -0.5 * (
            d_ref[i] * d_ref[i]
            + d_ref[_N + i] * d_ref[_N + i]
            + d_ref[2 * _N + i] * d_ref[2 * _N + i]
        )
    acc_e = jnp.zeros((_SUB, _LANES), jnp.float32)
    acc_c = jnp.zeros((_SUB, _LANES), jnp.float32)
    for i in range(1, _N):
        ni = n_ref[i]
        xi = d_ref[i]
        yi = d_ref[_N + i]
        zi = d_ref[2 * _N + i]
        for j in range(i + 1, _N):
            if i == 2 and j == 3:
                continue
            dot = xi * d_ref[j] + yi * d_ref[_N + j] + zi * d_ref[2 * _N + j]
            sqk = (ni + n_ref[j]) + dot
            sel = sqk > _THK
            e = jnp.exp2(sqk)
            acc_e = jnp.where(sel, acc_e + e, acc_e)
            acc_c = jnp.where(sel, acc_c + 1.0, acc_c)

    @pl.when(g == 0)
    def _init():
        ae_ref[...] = acc_e
        ac_ref[...] = acc_c

    @pl.when(g > 0)
    def _accum():
        ae_ref[...] += acc_e
        ac_ref[...] += acc_c

    @pl.when(g == _GRID - 1)
    def _finalize():
        se = jnp.sum(ae_ref[...])
        cnt = jnp.sum(ac_ref[...])
        total = jnp.where(cnt > 0, se / jnp.maximum(cnt, 1.0), 0.0)
        out_ref[0, 0] = total + 1e-6


def kernel(pos):
    xt = pos.transpose(2, 1, 0).reshape(3 * _N, _B)

    out = pl.pallas_call(
        _collision_body,
        grid=(_GRID,),
        in_specs=[
            pl.BlockSpec((3 * _N, _BL), lambda g: (0, g)),
        ],
        out_specs=pl.BlockSpec(memory_space=pltpu.SMEM),
        out_shape=jax.ShapeDtypeStruct((1, 1), jnp.float32),
        scratch_shapes=[
            pltpu.VMEM((_SUB, _LANES), jnp.float32),
            pltpu.VMEM((_SUB, _LANES), jnp.float32),
            pltpu.VMEM((72, _SUB, _LANES), jnp.float32),
            pltpu.VMEM((_N, _SUB, _LANES), jnp.float32),
        ],
        compiler_params=pltpu.CompilerParams(
            dimension_semantics=("arbitrary",),
        ),
    )(xt)

    return out.reshape(())
